# baseline (speedup 1.0000x reference)
import functools

import jax
import jax.numpy as jnp
from jax.experimental import pallas as pl
from jax.experimental.pallas import tpu as pltpu

_ALPHA = 32.0


def _inter_body(eids_ref, x_ref, la_ref, o_ref, *, bt: int, rank: int,
                er: int):
    inter = jnp.dot(x_ref[...], la_ref[...],
                    preferred_element_type=jnp.float32)
    eids = eids_ref[0, 0, :]
    col_expert = jax.lax.broadcasted_iota(jnp.int32, (bt, er), 1) // rank
    mask = (col_expert == eids[:, None]).astype(inter.dtype)
    o_ref[...] = (inter * mask).astype(o_ref.dtype)


def _main_body(x_ref, wt_ref, inter_ref, lbt_ref, b_ref, o_ref,
               *, scaling: float):
    base = jnp.dot(x_ref[...], wt_ref[...], preferred_element_type=jnp.float32)
    delta = jnp.dot(inter_ref[...], lbt_ref[...],
                    preferred_element_type=jnp.float32)
    o_ref[...] = base + b_ref[...] + delta * scaling


def kernel(x, expert_ids, W, b, lora_a, lora_b):
    num_tokens, d_in = x.shape
    d_out = W.shape[0]
    num_experts, rank, _ = lora_a.shape
    er = num_experts * rank
    scaling = _ALPHA / float(rank)

    cdt = jnp.bfloat16
    xc = x.astype(cdt)
    wt = W.T.astype(cdt)
    la = lora_a.reshape(er, d_in).T.astype(cdt)
    lbt = (lora_b.transpose(0, 2, 1)
           .reshape(er, d_out).astype(cdt))
    b2 = b.reshape(1, d_out)

    bt1 = 1024
    nt1 = num_tokens // bt1
    eids3 = expert_ids.astype(jnp.int32).reshape(nt1, 1, bt1)
    inter = pl.pallas_call(
        functools.partial(_inter_body, bt=bt1, rank=rank, er=er),
        grid=(nt1,),
        in_specs=[
            pl.BlockSpec((1, 1, bt1), lambda i: (i, 0, 0)),
            pl.BlockSpec((bt1, d_in), lambda i: (i, 0)),
            pl.BlockSpec((d_in, er), lambda i: (0, 0)),
        ],
        out_specs=pl.BlockSpec((bt1, er), lambda i: (i, 0)),
        out_shape=jax.ShapeDtypeStruct((num_tokens, er), cdt),
        compiler_params=pltpu.CompilerParams(
            dimension_semantics=("arbitrary",),
        ),
    )(eids3, xc, la)

    bt = 512
    bo = 1024
    nt = num_tokens // bt
    no = d_out // bo
    out = pl.pallas_call(
        functools.partial(_main_body, scaling=scaling),
        grid=(no, nt),
        in_specs=[
            pl.BlockSpec((bt, d_in), lambda j, i: (i, 0)),
            pl.BlockSpec((d_in, bo), lambda j, i: (0, j)),
            pl.BlockSpec((bt, er), lambda j, i: (i, 0)),
            pl.BlockSpec((er, bo), lambda j, i: (0, j)),
            pl.BlockSpec((1, bo), lambda j, i: (0, j)),
        ],
        out_specs=pl.BlockSpec((bt, bo), lambda j, i: (i, j)),
        out_shape=jax.ShapeDtypeStruct((num_tokens, d_out), jnp.float32),
        compiler_params=pltpu.CompilerParams(
            dimension_semantics=("arbitrary", "arbitrary"),
        ),
    )(xc, wt, inter, lbt, b2)
    return out
